# baseline (speedup 1.0000x reference)
import functools

import jax
import jax.numpy as jnp
from jax import lax
from jax.experimental import pallas as pl
from jax.experimental.pallas import tpu as pltpu
from jax.experimental.pallas import tpu_sc as plsc

_N = 4194304
_COLS = 1024
_ROWS = _N // _COLS

_SC_ROWS = 512
_TC_ROWS = _ROWS - _SC_ROWS
_M = _SC_ROWS * _COLS

_NC = 2
_NS = 16
_NW = _NC * _NS
_PER_W = _M // _NW
_NCHUNK = 2
_CHUNK = _PER_W // _NCHUNK
_L = 16
_UNROLL = 8

_mesh = plsc.VectorSubcoreMesh(core_axis_name="c", subcore_axis_name="s")


@functools.partial(
    pl.kernel,
    mesh=_mesh,
    out_type=jax.ShapeDtypeStruct((_M,), jnp.float32),
    scratch_types=[
        pltpu.VMEM((2, _CHUNK), jnp.float32),
        pltpu.VMEM((_L,), jnp.float32),
        pltpu.VMEM((_L,), jnp.float32),
        pltpu.SemaphoreType.DMA,
        pltpu.SemaphoreType.DMA,
        pltpu.SemaphoreType.DMA,
        pltpu.SemaphoreType.DMA,
    ],
)
def _affine_sc(x_hbm, scale_hbm, shift_hbm, out_hbm, buf, scv, shv,
               in_sem0, in_sem1, out_sem0, out_sem1):
    wid = lax.axis_index("s") * _NC + lax.axis_index("c")
    base = wid * _PER_W
    xbase = _TC_ROWS * _COLS + base

    pltpu.sync_copy(scale_hbm, scv)
    pltpu.sync_copy(shift_hbm, shv)
    s = scv[...]
    t = shv[...]

    in_sems = (in_sem0, in_sem1)
    out_sems = (out_sem0, out_sem1)
    cp_in = [None, None]
    cp_out = [None, None]

    cp_in[0] = pltpu.async_copy(
        x_hbm.at[pl.ds(xbase, _CHUNK)], buf.at[0], in_sems[0])

    for i in range(_NCHUNK):
        p = i % 2
        if i + 1 < _NCHUNK:
            q = (i + 1) % 2
            if i >= 1:
                cp_out[q].wait()
            cp_in[q] = pltpu.async_copy(
                x_hbm.at[pl.ds(xbase + (i + 1) * _CHUNK, _CHUNK)],
                buf.at[q], in_sems[q])
        cp_in[p].wait()

        def body(j, _, p=p):
            b0 = j * (_L * _UNROLL)
            for u in range(_UNROLL):
                sl = pl.ds(b0 + u * _L, _L)
                buf[p, sl] = buf[p, sl] * s + t
            return 0

        lax.fori_loop(0, _CHUNK // (_L * _UNROLL), body, 0)

        cp_out[p] = pltpu.async_copy(
            buf.at[p], out_hbm.at[pl.ds(base + i * _CHUNK, _CHUNK)],
            out_sems[p])

    for i in range(min(2, _NCHUNK)):
        cp_out[(_NCHUNK - 1 - i) % 2].wait()


_BR = 256


def _affine_tc_body(s_ref, t_ref, x_ref, o_ref):
    o_ref[...] = x_ref[...] * s_ref[0] + t_ref[0]


_affine_tc = pl.pallas_call(
    _affine_tc_body,
    grid=(_TC_ROWS // _BR,),
    in_specs=[
        pl.BlockSpec(memory_space=pltpu.SMEM),
        pl.BlockSpec(memory_space=pltpu.SMEM),
        pl.BlockSpec((_BR, _COLS), lambda i: (i, 0)),
    ],
    out_specs=pl.BlockSpec((_BR, _COLS), lambda i: (i, 0)),
    out_shape=jax.ShapeDtypeStruct((_ROWS, _COLS), jnp.float32),
)


def kernel(x, head, scale, shift):
    del head
    x2 = x.reshape(_ROWS, _COLS)
    scv = jnp.full((_L,), scale, dtype=jnp.float32)
    shv = jnp.full((_L,), shift, dtype=jnp.float32)
    sc_out = _affine_sc(x, scv, shv)
    tc_out = _affine_tc(scale.reshape(1), shift.reshape(1), x2)
    out = lax.dynamic_update_slice(
        tc_out, sc_out.reshape(_SC_ROWS, _COLS), (_TC_ROWS, 0))
    return out.reshape(-1)


# speedup vs baseline: 1.9250x; 1.9250x over previous
import functools

import jax
import jax.numpy as jnp
from jax import lax
from jax.experimental import pallas as pl
from jax.experimental.pallas import tpu as pltpu
from jax.experimental.pallas import tpu_sc as plsc

_N = 4194304

_M = 262144
_TC_N = _N - _M

_NC = 2
_NS = 16
_NW = _NC * _NS
_PER_W = _M // _NW
_NCHUNK = 2
_CHUNK = _PER_W // _NCHUNK
_L = 16
_UNROLL = 8

_mesh = plsc.VectorSubcoreMesh(core_axis_name="c", subcore_axis_name="s")


@functools.partial(
    pl.kernel,
    mesh=_mesh,
    out_type=jax.ShapeDtypeStruct((_M,), jnp.float32),
    scratch_types=[
        pltpu.VMEM((2, _CHUNK), jnp.float32),
        pltpu.VMEM((_L,), jnp.float32),
        pltpu.VMEM((_L,), jnp.float32),
        pltpu.SemaphoreType.DMA,
        pltpu.SemaphoreType.DMA,
        pltpu.SemaphoreType.DMA,
        pltpu.SemaphoreType.DMA,
    ],
)
def _affine_sc(x_hbm, scale_hbm, shift_hbm, out_hbm, buf, scv, shv,
               in_sem0, in_sem1, out_sem0, out_sem1):
    wid = lax.axis_index("s") * _NC + lax.axis_index("c")
    base = wid * _PER_W
    xbase = _TC_N + base

    pltpu.sync_copy(scale_hbm, scv)
    pltpu.sync_copy(shift_hbm, shv)
    s = scv[...]
    t = shv[...]

    in_sems = (in_sem0, in_sem1)
    out_sems = (out_sem0, out_sem1)
    cp_in = [None, None]
    cp_out = [None, None]

    cp_in[0] = pltpu.async_copy(
        x_hbm.at[pl.ds(xbase, _CHUNK)], buf.at[0], in_sems[0])

    for i in range(_NCHUNK):
        p = i % 2
        if i + 1 < _NCHUNK:
            q = (i + 1) % 2
            if i >= 1:
                cp_out[q].wait()
            cp_in[q] = pltpu.async_copy(
                x_hbm.at[pl.ds(xbase + (i + 1) * _CHUNK, _CHUNK)],
                buf.at[q], in_sems[q])
        cp_in[p].wait()

        def body(j, _, p=p):
            b0 = j * (_L * _UNROLL)
            for u in range(_UNROLL):
                sl = pl.ds(b0 + u * _L, _L)
                buf[p, sl] = buf[p, sl] * s + t
            return 0

        lax.fori_loop(0, _CHUNK // (_L * _UNROLL), body, 0)

        cp_out[p] = pltpu.async_copy(
            buf.at[p], out_hbm.at[pl.ds(base + i * _CHUNK, _CHUNK)],
            out_sems[p])

    for i in range(min(2, _NCHUNK)):
        cp_out[(_NCHUNK - 1 - i) % 2].wait()


_BLK = 262144


def _affine_tc_body(s_ref, t_ref, x_ref, o_ref):
    o_ref[...] = x_ref[...] * s_ref[0] + t_ref[0]


_affine_tc = pl.pallas_call(
    _affine_tc_body,
    grid=(_TC_N // _BLK,),
    in_specs=[
        pl.BlockSpec(memory_space=pltpu.SMEM),
        pl.BlockSpec(memory_space=pltpu.SMEM),
        pl.BlockSpec((_BLK,), lambda i: (i,)),
    ],
    out_specs=pl.BlockSpec((_BLK,), lambda i: (i,)),
    out_shape=jax.ShapeDtypeStruct((_N,), jnp.float32),
)


def kernel(x, head, scale, shift):
    del head
    scv = jnp.full((_L,), scale, dtype=jnp.float32)
    shv = jnp.full((_L,), shift, dtype=jnp.float32)
    sc_out = _affine_sc(x, scv, shv)
    tc_out = _affine_tc(scale.reshape(1), shift.reshape(1), x)
    return lax.dynamic_update_slice(tc_out, sc_out, (_TC_N,))


# speedup vs baseline: 4.8916x; 2.5411x over previous
import jax
import jax.numpy as jnp
from jax.experimental import pallas as pl
from jax.experimental.pallas import tpu as pltpu

_N = 4194304
_BLK = 524288


def _affine_tc_body(s_ref, t_ref, x_ref, o_ref):
    o_ref[...] = x_ref[...] * s_ref[0] + t_ref[0]


_affine_tc = pl.pallas_call(
    _affine_tc_body,
    grid=(_N // _BLK,),
    in_specs=[
        pl.BlockSpec(memory_space=pltpu.SMEM),
        pl.BlockSpec(memory_space=pltpu.SMEM),
        pl.BlockSpec((_BLK,), lambda i: (i,)),
    ],
    out_specs=pl.BlockSpec((_BLK,), lambda i: (i,)),
    out_shape=jax.ShapeDtypeStruct((_N,), jnp.float32),
)


def kernel(x, head, scale, shift):
    del head
    return _affine_tc(scale.reshape(1), shift.reshape(1), x)
